# Initial kernel scaffold; baseline (speedup 1.0000x reference)
import jax
import jax.numpy as jnp
from jax.experimental import pallas as pl


def kernel(hidden_states, Wg, W_gate, W_up, W_down):
    raise NotImplementedError("write your pallas kernel here")


# speedup vs baseline: 1.2958x; 1.2958x over previous
import jax
import jax.numpy as jnp
from jax.experimental import pallas as pl
from jax.experimental.pallas import tpu as pltpu

NUM_EXPERTS = 8
TOP_K = 2
MODEL_DIM = 1024
EXPERT_DIM = 2048
TOKENS = 2048
TBLK = 512


def _dense_body(x_ref, wg_ref, wgate_ref, wup_ref, wdown_ref,
                out_ref, logits_ref, we_ref, acc_ref, logits_scr):
    e = pl.program_id(0)
    t = pl.program_id(1)

    xb = x_ref[...]

    @pl.when(e == 0)
    def _router():
        logits = jax.lax.dot_general(
            xb, wg_ref[...], (((1,), (1,)), ((), ())),
            preferred_element_type=jnp.float32)
        logits_b = logits.astype(jnp.bfloat16)
        logits_scr[pl.ds(t * TBLK, TBLK), :] = logits_b
        p = jax.nn.softmax(logits_b.astype(jnp.float32), axis=1)
        eidx = jax.lax.broadcasted_iota(jnp.int32, p.shape, 1)
        m1 = jnp.max(p, axis=1, keepdims=True)
        i1 = jnp.min(jnp.where(p == m1, eidx, NUM_EXPERTS), axis=1,
                     keepdims=True)
        p_no1 = jnp.where(eidx == i1, -jnp.inf, p)
        m2 = jnp.max(p_no1, axis=1, keepdims=True)
        i2 = jnp.min(jnp.where(p_no1 == m2, eidx, NUM_EXPERTS), axis=1,
                     keepdims=True)
        selected = (eidx == i1) | (eidx == i2)
        denom = jnp.sum(jnp.where(selected, p, 0.0), axis=1, keepdims=True)
        we = jnp.where(selected, p, 0.0) / denom
        we_ref[pl.ds(t * TBLK, TBLK), :] = we.astype(jnp.bfloat16)

    wgate = wgate_ref[0]
    wup = wup_ref[0]
    wdown = wdown_ref[0]

    g = jax.lax.dot_general(xb, wgate, (((1,), (1,)), ((), ())),
                            preferred_element_type=jnp.float32)
    u = jax.lax.dot_general(xb, wup, (((1,), (1,)), ((), ())),
                            preferred_element_type=jnp.float32)
    gb = g.astype(jnp.bfloat16).astype(jnp.float32)
    ub = u.astype(jnp.bfloat16)
    h = ((gb * jax.nn.sigmoid(gb)).astype(jnp.bfloat16) * ub)
    y = jax.lax.dot_general(h, wdown, (((1,), (1,)), ((), ())),
                            preferred_element_type=jnp.float32)
    we_full = we_ref[pl.ds(t * TBLK, TBLK), :].astype(jnp.float32)
    eidx2 = jax.lax.broadcasted_iota(jnp.int32, we_full.shape, 1)
    we_col = jnp.sum(jnp.where(eidx2 == e, we_full, 0.0), axis=1,
                     keepdims=True)
    yw = (y.astype(jnp.bfloat16).astype(jnp.float32) *
          we_col).astype(jnp.bfloat16)

    @pl.when(e == 0)
    def _init():
        acc_ref[pl.ds(t * TBLK, TBLK), :] = yw

    @pl.when(e > 0)
    def _acc():
        acc_ref[pl.ds(t * TBLK, TBLK), :] = (
            acc_ref[pl.ds(t * TBLK, TBLK), :] + yw)

    out_ref[...] = acc_ref[pl.ds(t * TBLK, TBLK), :]
    logits_ref[...] = logits_scr[pl.ds(t * TBLK, TBLK), :]


def kernel(hidden_states, Wg, W_gate, W_up, W_down):
    B, S, D = hidden_states.shape
    x = hidden_states.reshape(S, D)

    grid = (NUM_EXPERTS, S // TBLK)
    out, logits = pl.pallas_call(
        _dense_body,
        grid=grid,
        in_specs=[
            pl.BlockSpec((TBLK, D), lambda e, t: (t, 0)),
            pl.BlockSpec((NUM_EXPERTS, D), lambda e, t: (0, 0)),
            pl.BlockSpec((1, EXPERT_DIM, D), lambda e, t: (e, 0, 0)),
            pl.BlockSpec((1, EXPERT_DIM, D), lambda e, t: (e, 0, 0)),
            pl.BlockSpec((1, D, EXPERT_DIM), lambda e, t: (e, 0, 0)),
        ],
        out_specs=[
            pl.BlockSpec((TBLK, D), lambda e, t: (t, 0)),
            pl.BlockSpec((TBLK, NUM_EXPERTS), lambda e, t: (t, 0)),
        ],
        out_shape=[
            jax.ShapeDtypeStruct((S, D), jnp.bfloat16),
            jax.ShapeDtypeStruct((S, NUM_EXPERTS), jnp.bfloat16),
        ],
        scratch_shapes=[
            pltpu.VMEM((S, NUM_EXPERTS), jnp.bfloat16),
            pltpu.VMEM((S, D), jnp.bfloat16),
            pltpu.VMEM((S, NUM_EXPERTS), jnp.bfloat16),
        ],
    )(x, Wg, W_gate, W_up, W_down)

    return out.reshape(B, S, D), logits
